# Initial kernel scaffold; baseline (speedup 1.0000x reference)
import jax
import jax.numpy as jnp
from jax.experimental import pallas as pl


def kernel(hv, edge_index, he, W_msg, b_msg, W_ih, W_hh, b_ih, b_hh):
    raise NotImplementedError("write your pallas kernel here")


# speedup vs baseline: 7.5435x; 7.5435x over previous
import functools

import jax
import jax.numpy as jnp
from jax import lax
from jax.experimental import pallas as pl
from jax.experimental.pallas import tpu as pltpu
from jax.experimental.pallas import tpu_sc as plsc

_NUM_ROUNDS = 2
_D = 128
_N = 10000
_E = 320000

_NC = 2
_NS = 16
_NW = _NC * _NS
_EPW = _E // _NW
_CH = 80
_NCHUNK = _EPW // _CH
_NPAD = 10240
_RPS = _NPAD // _NS

def _sc_segsum_h_body(h_hbm, src_hbm, dst_hbm, zrows_hbm, out_hbm,
                      sidx, didx, rows, acc, sem):
    cid = lax.axis_index("c")
    sid = lax.axis_index("s")
    w = cid * _NS + sid
    pltpu.sync_copy(zrows_hbm, rows)

    @pl.loop(0, _RPS // _CH)
    def _(j):
        pltpu.sync_copy(rows, acc.at[pl.ds(sid * _RPS + j * _CH, _CH)])

    pltpu.sync_copy(src_hbm.at[w], sidx)
    pltpu.sync_copy(dst_hbm.at[w], didx)
    plsc.subcore_barrier()

    @pl.loop(0, _NCHUNK)
    def _(c):
        pltpu.async_copy(h_hbm.at[sidx.at[c]], rows, sem).wait()
        pltpu.sync_copy(rows, acc.at[didx.at[c]], add=True)

    plsc.subcore_barrier()

    @pl.loop(0, _RPS // _CH)
    def _(j):
        pltpu.sync_copy(acc.at[pl.ds(sid * _RPS + j * _CH, _CH)], rows)
        pltpu.sync_copy(
            rows, out_hbm.at[pl.ds(cid * _NPAD + sid * _RPS + j * _CH, _CH)])


def _sc_segsum_he_body(he_hbm, dst_hbm, zrows_hbm, out_he_hbm,
                       didx, vals, acc):
    cid = lax.axis_index("c")
    sid = lax.axis_index("s")
    w = cid * _NS + sid
    pltpu.sync_copy(zrows_hbm, vals)

    @pl.loop(0, _RPS // _CH)
    def _(j):
        pltpu.sync_copy(vals, acc.at[pl.ds(sid * _RPS + j * _CH, _CH)])

    pltpu.sync_copy(dst_hbm.at[w], didx)
    plsc.subcore_barrier()

    @pl.loop(0, _NCHUNK)
    def _(c):
        pltpu.sync_copy(he_hbm.at[pl.ds(w * _EPW + c * _CH, _CH)], vals)
        pltpu.sync_copy(vals, acc.at[didx.at[c]], add=True)

    plsc.subcore_barrier()

    @pl.loop(0, _RPS // _CH)
    def _(j):
        pltpu.sync_copy(acc.at[pl.ds(sid * _RPS + j * _CH, _CH)], vals)
        pltpu.sync_copy(
            vals, out_he_hbm.at[pl.ds(cid * _NPAD + sid * _RPS + j * _CH, _CH)])


def _sc_deg_body(dst_hbm, zrows_hbm, ones_hbm, out_deg_hbm,
                 didx, vals, ones_v, dacc):
    cid = lax.axis_index("c")
    sid = lax.axis_index("s")
    w = cid * _NS + sid
    pltpu.sync_copy(zrows_hbm, vals)

    @pl.loop(0, _RPS // _CH)
    def _(j):
        pltpu.sync_copy(vals, dacc.at[pl.ds(sid * _RPS + j * _CH, _CH)])

    pltpu.sync_copy(dst_hbm.at[w], didx)
    pltpu.sync_copy(ones_hbm, ones_v)
    plsc.subcore_barrier()

    @pl.loop(0, _NCHUNK)
    def _(c):
        pltpu.sync_copy(ones_v, dacc.at[didx.at[c]], add=True)

    plsc.subcore_barrier()

    @pl.loop(0, _RPS // _CH)
    def _(j):
        pltpu.sync_copy(dacc.at[pl.ds(sid * _RPS + j * _CH, _CH)], vals)
        pltpu.sync_copy(
            vals, out_deg_hbm.at[pl.ds(cid * _NPAD + sid * _RPS + j * _CH, _CH)])


@functools.lru_cache(maxsize=None)
def _sc_kernels():
    mesh = plsc.VectorSubcoreMesh(core_axis_name="c", subcore_axis_name="s")
    segsum_h = pl.kernel(
        _sc_segsum_h_body,
        out_type=jax.ShapeDtypeStruct((_NC * _NPAD, _D), jnp.float32),
        mesh=mesh,
        scratch_types=[
            pltpu.VMEM((_NCHUNK, _CH), jnp.int32),
            pltpu.VMEM((_NCHUNK, _CH), jnp.int32),
            pltpu.VMEM((_CH, _D), jnp.float32),
            pltpu.VMEM_SHARED((_NPAD, _D), jnp.float32),
            pltpu.SemaphoreType.DMA,
        ],
    )
    segsum_he = pl.kernel(
        _sc_segsum_he_body,
        out_type=jax.ShapeDtypeStruct((_NC * _NPAD, _D), jnp.float32),
        mesh=mesh,
        scratch_types=[
            pltpu.VMEM((_NCHUNK, _CH), jnp.int32),
            pltpu.VMEM((_CH, _D), jnp.float32),
            pltpu.VMEM_SHARED((_NPAD, _D), jnp.float32),
        ],
    )
    deg_hist = pl.kernel(
        _sc_deg_body,
        out_type=jax.ShapeDtypeStruct((_NC * _NPAD, _D), jnp.float32),
        mesh=mesh,
        scratch_types=[
            pltpu.VMEM((_NCHUNK, _CH), jnp.int32),
            pltpu.VMEM((_CH, _D), jnp.float32),
            pltpu.VMEM((_CH, _D), jnp.float32),
            pltpu.VMEM_SHARED((_NPAD, _D), jnp.float32),
        ],
    )
    return segsum_h, segsum_he, deg_hist


_BN = 1024


def _bdot(x16, w16):
    return jnp.dot(x16, w16, preferred_element_type=jnp.float32)


def _lodot(x, w16):
    xh = x.astype(jnp.bfloat16)
    xl = (x - xh.astype(jnp.float32)).astype(jnp.bfloat16)
    return _bdot(xh, w16) + _bdot(xl, w16)


def _tc_update_body(h_ref, m0_ref, m1_ref, e0_ref, e1_ref, d0_ref, d1_ref,
                    w1t_ref, w2t_ref, w3t_ref, wiht_ref, whht_ref,
                    bmsg_ref, bih_ref, bhh_ref, out_ref):
    h = h_ref[...]
    h16 = h.astype(jnp.bfloat16)
    m = m0_ref[...] + m1_ref[...]
    hes = e0_ref[...] + e1_ref[...]
    deg = d0_ref[:, 0:1] + d1_ref[:, 0:1]
    w1t = w1t_ref[...].astype(jnp.bfloat16)
    w2t = w2t_ref[...].astype(jnp.bfloat16)
    w3t = w3t_ref[...].astype(jnp.bfloat16)
    wiht = wiht_ref[...].astype(jnp.bfloat16)
    whht = whht_ref[...].astype(jnp.bfloat16)
    a = (deg * (_bdot(h16, w1t) + bmsg_ref[0])
         + _lodot(m, w2t) + _lodot(hes, w3t))
    gi = _bdot(a.astype(jnp.bfloat16), wiht) + bih_ref[0]
    gh = _bdot(h16, whht) + bhh_ref[0]
    r = jax.nn.sigmoid(gi[:, :_D] + gh[:, :_D])
    z = jax.nn.sigmoid(gi[:, _D:2 * _D] + gh[:, _D:2 * _D])
    n = jnp.tanh(gi[:, 2 * _D:] + r * gh[:, 2 * _D:])
    out_ref[...] = (1.0 - z) * n + z * h


def _tc_update(h, m_acc, he_acc, deg_acc, w1t, w2t, w3t, wiht, whht,
               bmsg, bih, bhh):
    blk = lambda i: (i, 0)
    blk_hi = lambda i: (i + _NPAD // _BN, 0)
    full = lambda shape: pl.BlockSpec(shape, lambda i: (0, 0))
    return pl.pallas_call(
        _tc_update_body,
        grid=(_NPAD // _BN,),
        in_specs=[
            pl.BlockSpec((_BN, _D), blk),
            pl.BlockSpec((_BN, _D), blk),
            pl.BlockSpec((_BN, _D), blk_hi),
            pl.BlockSpec((_BN, _D), blk),
            pl.BlockSpec((_BN, _D), blk_hi),
            pl.BlockSpec((_BN, _D), blk),
            pl.BlockSpec((_BN, _D), blk_hi),
            full((_D, 2 * _D)), full((_D, 2 * _D)), full((_D, 2 * _D)),
            full((2 * _D, 3 * _D)), full((_D, 3 * _D)),
            full((1, 2 * _D)), full((1, 3 * _D)), full((1, 3 * _D)),
        ],
        out_specs=pl.BlockSpec((_BN, _D), blk),
        out_shape=jax.ShapeDtypeStruct((_NPAD, _D), jnp.float32),
    )(h, m_acc, m_acc, he_acc, he_acc, deg_acc, deg_acc,
      w1t, w2t, w3t, wiht, whht, bmsg, bih, bhh)


def kernel(hv, edge_index, he, W_msg, b_msg, W_ih, W_hh, b_ih, b_hh):
    src3 = edge_index[0].reshape(_NW, _NCHUNK, _CH)
    dst3 = edge_index[1].reshape(_NW, _NCHUNK, _CH)
    zrows = jnp.zeros((_CH, _D), jnp.float32)
    ones_v = jnp.ones((_CH, _D), jnp.float32)

    segsum_h, segsum_he, deg_hist = _sc_kernels()
    he_bf = he.astype(jnp.bfloat16).astype(jnp.float32)
    he_acc = segsum_he(he_bf, dst3, zrows)
    deg_acc = deg_hist(dst3, zrows, ones_v)

    h = jnp.pad(hv, ((0, _NPAD - _N), (0, 0)))
    for t in range(_NUM_ROUNDS):
        h_bf = h.astype(jnp.bfloat16).astype(jnp.float32)
        m_acc = segsum_h(h_bf, src3, dst3, zrows)
        h = _tc_update(
            h, m_acc, he_acc, deg_acc,
            W_msg[t, :, :_D].T, W_msg[t, :, _D:2 * _D].T, W_msg[t, :, 2 * _D:].T,
            W_ih[t].T, W_hh[t].T,
            b_msg[t][None], b_ih[t][None], b_hh[t][None])
    return h[:_N]


# speedup vs baseline: 9.3894x; 1.2447x over previous
import functools

import jax
import jax.numpy as jnp
from jax import lax
from jax.experimental import pallas as pl
from jax.experimental.pallas import tpu as pltpu
from jax.experimental.pallas import tpu_sc as plsc

_NUM_ROUNDS = 2
_D = 128
_N = 10000
_E = 320000

_NC = 2
_NS = 16
_NW = _NC * _NS
_EPW = _E // _NW
_CH = 80
_NCHUNK = _EPW // _CH
_NPAD = 10240
_RPS = _NPAD // _NS

_NR = 3
_NI = 6
_PRO = 5


def _sc_segsum_h_body(h_hbm, src_hbm, dst_hbm, zrows_hbm, out_hbm, *s):
    rows = s[0:_NR]
    sis = s[_NR:_NR + _NI]
    dis = s[_NR + _NI:_NR + 2 * _NI]
    acc = s[_NR + 2 * _NI]
    semg = s[_NR + 2 * _NI + 1:_NR + 2 * _NI + 1 + _NR]
    sems = s[_NR + 2 * _NI + 1 + _NR:_NR + 2 * _NI + 1 + 2 * _NR]
    semi = s[_NR + 2 * _NI + 1 + 2 * _NR:]
    cid = lax.axis_index("c")
    sid = lax.axis_index("s")
    w = cid * _NS + sid
    base = w * _EPW

    pltpu.sync_copy(zrows_hbm, rows[0])

    @pl.loop(0, _RPS // _CH)
    def _(j):
        pltpu.sync_copy(rows[0], acc.at[pl.ds(sid * _RPS + j * _CH, _CH)])

    plsc.subcore_barrier()

    def idx_issue(c, b6):
        pltpu.async_copy(src_hbm.at[pl.ds(base + c * _CH, _CH)], sis[b6],
                         semi[b6])
        pltpu.async_copy(dst_hbm.at[pl.ds(base + c * _CH, _CH)], dis[b6],
                         semi[b6])

    def idx_wait(b6):
        pltpu.make_async_copy(src_hbm.at[pl.ds(0, _CH)], sis[b6],
                              semi[b6]).wait()
        pltpu.make_async_copy(dst_hbm.at[pl.ds(0, _CH)], dis[b6],
                              semi[b6]).wait()

    def g_issue(b6, b3):
        pltpu.async_copy(h_hbm.at[sis[b6]], rows[b3], semg[b3])

    def g_wait(b3):
        pltpu.make_async_copy(h_hbm.at[pl.ds(0, _CH)], rows[b3],
                              semg[b3]).wait()

    def s_issue(b3, b6):
        pltpu.async_copy(rows[b3], acc.at[dis[b6]], sems[b3], add=True)

    def s_wait(b3):
        pltpu.make_async_copy(rows[b3], acc.at[pl.ds(0, _CH)],
                              sems[b3]).wait()

    for k in range(_PRO):
        idx_issue(k, k % _NI)
        idx_wait(k % _NI)
        g_issue(k % _NI, k % _NR)
        g_wait(k % _NR)
        pltpu.sync_copy(rows[k % _NR], acc.at[dis[k % _NI]], add=True)

    for k in range(_PRO, _PRO + 4):
        idx_issue(k, k % _NI)
    for k in range(_PRO, _PRO + 2):
        idx_wait(k % _NI)
        g_issue(k % _NI, k % _NR)

    @pl.loop(0, (_NCHUNK - _PRO) // _NI)
    def _(c0):
        for j in range(_NI):
            b3 = (_PRO + j) % _NR
            b6 = (_PRO + j) % _NI
            c = _PRO + c0 * _NI + j
            g_wait(b3)
            s_issue(b3, b6)

            @pl.when(c + 4 < _NCHUNK)
            def _():
                idx_issue(c + 4, (b6 + 4) % _NI)

            @pl.when(c >= _PRO + 1)
            def _():
                s_wait((b3 + 2) % _NR)

            @pl.when(c + 2 < _NCHUNK)
            def _():
                idx_wait((b6 + 2) % _NI)
                g_issue((b6 + 2) % _NI, (b3 + 2) % _NR)

    s_wait((_NCHUNK - 1) % _NR)
    plsc.subcore_barrier()

    @pl.loop(0, _RPS // _CH)
    def _(j):
        pltpu.sync_copy(acc.at[pl.ds(sid * _RPS + j * _CH, _CH)], rows[0])
        pltpu.sync_copy(
            rows[0],
            out_hbm.at[pl.ds(cid * _NPAD + sid * _RPS + j * _CH, _CH)])


def _sc_segsum_he_body(he_hbm, dst_hbm, zrows_hbm, out_he_hbm,
                       didx, vals, acc):
    cid = lax.axis_index("c")
    sid = lax.axis_index("s")
    w = cid * _NS + sid
    pltpu.sync_copy(zrows_hbm, vals)

    @pl.loop(0, _RPS // _CH)
    def _(j):
        pltpu.sync_copy(vals, acc.at[pl.ds(sid * _RPS + j * _CH, _CH)])

    pltpu.sync_copy(dst_hbm.at[w], didx)
    plsc.subcore_barrier()

    @pl.loop(0, _NCHUNK)
    def _(c):
        pltpu.sync_copy(he_hbm.at[pl.ds(w * _EPW + c * _CH, _CH)], vals)
        pltpu.sync_copy(vals, acc.at[didx.at[c]], add=True)

    plsc.subcore_barrier()

    @pl.loop(0, _RPS // _CH)
    def _(j):
        pltpu.sync_copy(acc.at[pl.ds(sid * _RPS + j * _CH, _CH)], vals)
        pltpu.sync_copy(
            vals, out_he_hbm.at[pl.ds(cid * _NPAD + sid * _RPS + j * _CH, _CH)])


def _sc_deg_body(dst_hbm, zrows_hbm, ones_hbm, out_deg_hbm,
                 didx, vals, ones_v, dacc):
    cid = lax.axis_index("c")
    sid = lax.axis_index("s")
    w = cid * _NS + sid
    pltpu.sync_copy(zrows_hbm, vals)

    @pl.loop(0, _RPS // _CH)
    def _(j):
        pltpu.sync_copy(vals, dacc.at[pl.ds(sid * _RPS + j * _CH, _CH)])

    pltpu.sync_copy(dst_hbm.at[w], didx)
    pltpu.sync_copy(ones_hbm, ones_v)
    plsc.subcore_barrier()

    @pl.loop(0, _NCHUNK)
    def _(c):
        pltpu.sync_copy(ones_v, dacc.at[didx.at[c]], add=True)

    plsc.subcore_barrier()

    @pl.loop(0, _RPS // _CH)
    def _(j):
        pltpu.sync_copy(dacc.at[pl.ds(sid * _RPS + j * _CH, _CH)], vals)
        pltpu.sync_copy(
            vals, out_deg_hbm.at[pl.ds(cid * _NPAD + sid * _RPS + j * _CH, _CH)])


@functools.lru_cache(maxsize=None)
def _sc_kernels():
    mesh = plsc.VectorSubcoreMesh(core_axis_name="c", subcore_axis_name="s")
    segsum_h = pl.kernel(
        _sc_segsum_h_body,
        out_type=jax.ShapeDtypeStruct((_NC * _NPAD, _D), jnp.float32),
        mesh=mesh,
        scratch_types=(
            [pltpu.VMEM((_CH, _D), jnp.float32) for _ in range(_NR)]
            + [pltpu.VMEM((_CH,), jnp.int32) for _ in range(_NI)]
            + [pltpu.VMEM((_CH,), jnp.int32) for _ in range(_NI)]
            + [pltpu.VMEM_SHARED((_NPAD, _D), jnp.float32)]
            + [pltpu.SemaphoreType.DMA for _ in range(2 * _NR + _NI)]
        ),
    )
    segsum_he = pl.kernel(
        _sc_segsum_he_body,
        out_type=jax.ShapeDtypeStruct((_NC * _NPAD, _D), jnp.float32),
        mesh=mesh,
        scratch_types=[
            pltpu.VMEM((_NCHUNK, _CH), jnp.int32),
            pltpu.VMEM((_CH, _D), jnp.float32),
            pltpu.VMEM_SHARED((_NPAD, _D), jnp.float32),
        ],
    )
    deg_hist = pl.kernel(
        _sc_deg_body,
        out_type=jax.ShapeDtypeStruct((_NC * _NPAD, _D), jnp.float32),
        mesh=mesh,
        scratch_types=[
            pltpu.VMEM((_NCHUNK, _CH), jnp.int32),
            pltpu.VMEM((_CH, _D), jnp.float32),
            pltpu.VMEM((_CH, _D), jnp.float32),
            pltpu.VMEM_SHARED((_NPAD, _D), jnp.float32),
        ],
    )
    return segsum_h, segsum_he, deg_hist


_BN = 1024


def _bdot(x16, w16):
    return jnp.dot(x16, w16, preferred_element_type=jnp.float32)


def _lodot(x, w16):
    xh = x.astype(jnp.bfloat16)
    xl = (x - xh.astype(jnp.float32)).astype(jnp.bfloat16)
    return _bdot(xh, w16) + _bdot(xl, w16)


def _tc_update_body(h_ref, m0_ref, m1_ref, e0_ref, e1_ref, d0_ref, d1_ref,
                    w1t_ref, w2t_ref, w3t_ref, wiht_ref, whht_ref,
                    bmsg_ref, bih_ref, bhh_ref, out_ref):
    h = h_ref[...]
    h16 = h.astype(jnp.bfloat16)
    m = m0_ref[...] + m1_ref[...]
    hes = e0_ref[...] + e1_ref[...]
    deg = d0_ref[:, 0:1] + d1_ref[:, 0:1]
    w1t = w1t_ref[...].astype(jnp.bfloat16)
    w2t = w2t_ref[...].astype(jnp.bfloat16)
    w3t = w3t_ref[...].astype(jnp.bfloat16)
    wiht = wiht_ref[...].astype(jnp.bfloat16)
    whht = whht_ref[...].astype(jnp.bfloat16)
    a = (deg * (_bdot(h16, w1t) + bmsg_ref[0])
         + _lodot(m, w2t) + _lodot(hes, w3t))
    gi = _bdot(a.astype(jnp.bfloat16), wiht) + bih_ref[0]
    gh = _bdot(h16, whht) + bhh_ref[0]
    r = jax.nn.sigmoid(gi[:, :_D] + gh[:, :_D])
    z = jax.nn.sigmoid(gi[:, _D:2 * _D] + gh[:, _D:2 * _D])
    n = jnp.tanh(gi[:, 2 * _D:] + r * gh[:, 2 * _D:])
    out_ref[...] = (1.0 - z) * n + z * h


def _tc_update(h, m_acc, he_acc, deg_acc, w1t, w2t, w3t, wiht, whht,
               bmsg, bih, bhh):
    blk = lambda i: (i, 0)
    blk_hi = lambda i: (i + _NPAD // _BN, 0)
    full = lambda shape: pl.BlockSpec(shape, lambda i: (0, 0))
    return pl.pallas_call(
        _tc_update_body,
        grid=(_NPAD // _BN,),
        in_specs=[
            pl.BlockSpec((_BN, _D), blk),
            pl.BlockSpec((_BN, _D), blk),
            pl.BlockSpec((_BN, _D), blk_hi),
            pl.BlockSpec((_BN, _D), blk),
            pl.BlockSpec((_BN, _D), blk_hi),
            pl.BlockSpec((_BN, _D), blk),
            pl.BlockSpec((_BN, _D), blk_hi),
            full((_D, 2 * _D)), full((_D, 2 * _D)), full((_D, 2 * _D)),
            full((2 * _D, 3 * _D)), full((_D, 3 * _D)),
            full((1, 2 * _D)), full((1, 3 * _D)), full((1, 3 * _D)),
        ],
        out_specs=pl.BlockSpec((_BN, _D), blk),
        out_shape=jax.ShapeDtypeStruct((_NPAD, _D), jnp.float32),
    )(h, m_acc, m_acc, he_acc, he_acc, deg_acc, deg_acc,
      w1t, w2t, w3t, wiht, whht, bmsg, bih, bhh)


def kernel(hv, edge_index, he, W_msg, b_msg, W_ih, W_hh, b_ih, b_hh):
    src = edge_index[0]
    dst = edge_index[1]
    dst3 = dst.reshape(_NW, _NCHUNK, _CH)
    zrows = jnp.zeros((_CH, _D), jnp.float32)
    ones_v = jnp.ones((_CH, _D), jnp.float32)

    segsum_h, segsum_he, deg_hist = _sc_kernels()
    he_bf = he.astype(jnp.bfloat16).astype(jnp.float32)
    he_acc = segsum_he(he_bf, dst3, zrows)
    deg_acc = deg_hist(dst3, zrows, ones_v)

    h = jnp.pad(hv, ((0, _NPAD - _N), (0, 0)))
    for t in range(_NUM_ROUNDS):
        h_bf = h.astype(jnp.bfloat16).astype(jnp.float32)
        m_acc = segsum_h(h_bf, src, dst, zrows)
        h = _tc_update(
            h, m_acc, he_acc, deg_acc,
            W_msg[t, :, :_D].T, W_msg[t, :, _D:2 * _D].T, W_msg[t, :, 2 * _D:].T,
            W_ih[t].T, W_hh[t].T,
            b_msg[t][None], b_ih[t][None], b_hh[t][None])
    return h[:_N]


# speedup vs baseline: 11.4669x; 1.2213x over previous
import functools

import jax
import jax.numpy as jnp
from jax import lax
from jax.experimental import pallas as pl
from jax.experimental.pallas import tpu as pltpu
from jax.experimental.pallas import tpu_sc as plsc

_NUM_ROUNDS = 2
_D = 128
_N = 10000
_E = 320000

_NC = 2
_NS = 16
_NW = _NC * _NS
_EPW = _E // _NW
_CH = 80
_NCHUNK = _EPW // _CH
_NPAD = 10240
_RPS = _NPAD // _NS

_NR = 3
_NI = 6
_PRO = 5


def _sc_segsum_h_body(h_hbm, src_hbm, dst_hbm, zrows_hbm, out_hbm, *s):
    rows = s[0:_NR]
    sis = s[_NR:_NR + _NI]
    dis = s[_NR + _NI:_NR + 2 * _NI]
    acc = s[_NR + 2 * _NI]
    semg = s[_NR + 2 * _NI + 1:_NR + 2 * _NI + 1 + _NR]
    sems = s[_NR + 2 * _NI + 1 + _NR:_NR + 2 * _NI + 1 + 2 * _NR]
    semi = s[_NR + 2 * _NI + 1 + 2 * _NR:]
    cid = lax.axis_index("c")
    sid = lax.axis_index("s")
    w = cid * _NS + sid
    base = w * _EPW

    pltpu.sync_copy(zrows_hbm, rows[0])

    @pl.loop(0, _RPS // _CH)
    def _(j):
        pltpu.sync_copy(rows[0], acc.at[pl.ds(sid * _RPS + j * _CH, _CH)])

    plsc.subcore_barrier()

    def idx_issue(c, b6):
        pltpu.async_copy(src_hbm.at[pl.ds(base + c * _CH, _CH)], sis[b6],
                         semi[b6])
        pltpu.async_copy(dst_hbm.at[pl.ds(base + c * _CH, _CH)], dis[b6],
                         semi[b6])

    def idx_wait(b6):
        pltpu.make_async_copy(src_hbm.at[pl.ds(0, _CH)], sis[b6],
                              semi[b6]).wait()
        pltpu.make_async_copy(dst_hbm.at[pl.ds(0, _CH)], dis[b6],
                              semi[b6]).wait()

    def g_issue(b6, b3):
        pltpu.async_copy(h_hbm.at[sis[b6]], rows[b3], semg[b3])

    def g_wait(b3):
        pltpu.make_async_copy(h_hbm.at[pl.ds(0, _CH)], rows[b3],
                              semg[b3]).wait()

    def s_issue(b3, b6):
        pltpu.async_copy(rows[b3], acc.at[dis[b6]], sems[b3], add=True)

    def s_wait(b3):
        pltpu.make_async_copy(rows[b3], acc.at[pl.ds(0, _CH)],
                              sems[b3]).wait()

    for k in range(_PRO):
        idx_issue(k, k % _NI)
        idx_wait(k % _NI)
        g_issue(k % _NI, k % _NR)
        g_wait(k % _NR)
        pltpu.sync_copy(rows[k % _NR], acc.at[dis[k % _NI]], add=True)

    for k in range(_PRO, _PRO + 4):
        idx_issue(k, k % _NI)
    for k in range(_PRO, _PRO + 2):
        idx_wait(k % _NI)
        g_issue(k % _NI, k % _NR)

    @pl.loop(0, (_NCHUNK - _PRO) // _NI)
    def _(c0):
        for j in range(_NI):
            b3 = (_PRO + j) % _NR
            b6 = (_PRO + j) % _NI
            c = _PRO + c0 * _NI + j
            g_wait(b3)
            s_issue(b3, b6)

            @pl.when(c + 4 < _NCHUNK)
            def _():
                idx_issue(c + 4, (b6 + 4) % _NI)

            @pl.when(c >= _PRO + 1)
            def _():
                s_wait((b3 + 2) % _NR)

            @pl.when(c + 2 < _NCHUNK)
            def _():
                idx_wait((b6 + 2) % _NI)
                g_issue((b6 + 2) % _NI, (b3 + 2) % _NR)

    s_wait((_NCHUNK - 1) % _NR)
    plsc.subcore_barrier()

    @pl.loop(0, _RPS // _CH)
    def _(j):
        pltpu.sync_copy(acc.at[pl.ds(sid * _RPS + j * _CH, _CH)], rows[0])
        pltpu.sync_copy(
            rows[0],
            out_hbm.at[pl.ds(cid * _NPAD + sid * _RPS + j * _CH, _CH)])


def _sc_segsum_he_body(he_hbm, dst_hbm, zrows_hbm, out_he_hbm, *s):
    rows = s[0:_NR]
    dis = s[_NR:_NR + _NI]
    acc = s[_NR + _NI]
    semg = s[_NR + _NI + 1:_NR + _NI + 1 + _NR]
    sems = s[_NR + _NI + 1 + _NR:_NR + _NI + 1 + 2 * _NR]
    semi = s[_NR + _NI + 1 + 2 * _NR:]
    cid = lax.axis_index("c")
    sid = lax.axis_index("s")
    w = cid * _NS + sid
    base = w * _EPW

    pltpu.sync_copy(zrows_hbm, rows[0])

    @pl.loop(0, _RPS // _CH)
    def _(j):
        pltpu.sync_copy(rows[0], acc.at[pl.ds(sid * _RPS + j * _CH, _CH)])

    plsc.subcore_barrier()

    def idx_issue(c, b6):
        pltpu.async_copy(dst_hbm.at[pl.ds(base + c * _CH, _CH)], dis[b6],
                         semi[b6])

    def idx_wait(b6):
        pltpu.make_async_copy(dst_hbm.at[pl.ds(0, _CH)], dis[b6],
                              semi[b6]).wait()

    def v_issue(c, b3):
        pltpu.async_copy(he_hbm.at[pl.ds(base + c * _CH, _CH)], rows[b3],
                         semg[b3])

    def v_wait(b3):
        pltpu.make_async_copy(he_hbm.at[pl.ds(0, _CH)], rows[b3],
                              semg[b3]).wait()

    def s_issue(b3, b6):
        pltpu.async_copy(rows[b3], acc.at[dis[b6]], sems[b3], add=True)

    def s_wait(b3):
        pltpu.make_async_copy(rows[b3], acc.at[pl.ds(0, _CH)],
                              sems[b3]).wait()

    for k in range(_PRO):
        idx_issue(k, k % _NI)
        idx_wait(k % _NI)
        v_issue(k, k % _NR)
        v_wait(k % _NR)
        pltpu.sync_copy(rows[k % _NR], acc.at[dis[k % _NI]], add=True)

    for k in range(_PRO, _PRO + 4):
        idx_issue(k, k % _NI)
    for k in range(_PRO, _PRO + 2):
        idx_wait(k % _NI)
        v_issue(k, k % _NR)

    @pl.loop(0, (_NCHUNK - _PRO) // _NI)
    def _(c0):
        for j in range(_NI):
            b3 = (_PRO + j) % _NR
            b6 = (_PRO + j) % _NI
            c = _PRO + c0 * _NI + j
            v_wait(b3)
            s_issue(b3, b6)

            @pl.when(c + 4 < _NCHUNK)
            def _():
                idx_issue(c + 4, (b6 + 4) % _NI)

            @pl.when(c >= _PRO + 1)
            def _():
                s_wait((b3 + 2) % _NR)

            @pl.when(c + 2 < _NCHUNK)
            def _():
                idx_wait((b6 + 2) % _NI)
                v_issue(c + 2, (b3 + 2) % _NR)

    s_wait((_NCHUNK - 1) % _NR)
    plsc.subcore_barrier()

    @pl.loop(0, _RPS // _CH)
    def _(j):
        pltpu.sync_copy(acc.at[pl.ds(sid * _RPS + j * _CH, _CH)], rows[0])
        pltpu.sync_copy(
            rows[0],
            out_he_hbm.at[pl.ds(cid * _NPAD + sid * _RPS + j * _CH, _CH)])


def _sc_deg_body(dst_hbm, zrows_hbm, ones_hbm, out_deg_hbm,
                 didx, vals, ones_v, dacc):
    cid = lax.axis_index("c")
    sid = lax.axis_index("s")
    w = cid * _NS + sid
    pltpu.sync_copy(zrows_hbm, vals)

    @pl.loop(0, _RPS // _CH)
    def _(j):
        pltpu.sync_copy(vals, dacc.at[pl.ds(sid * _RPS + j * _CH, _CH)])

    pltpu.sync_copy(dst_hbm.at[w], didx)
    pltpu.sync_copy(ones_hbm, ones_v)
    plsc.subcore_barrier()

    @pl.loop(0, _NCHUNK)
    def _(c):
        pltpu.sync_copy(ones_v, dacc.at[didx.at[c]], add=True)

    plsc.subcore_barrier()

    @pl.loop(0, _RPS // _CH)
    def _(j):
        pltpu.sync_copy(dacc.at[pl.ds(sid * _RPS + j * _CH, _CH)], vals)
        pltpu.sync_copy(
            vals, out_deg_hbm.at[pl.ds(cid * _NPAD + sid * _RPS + j * _CH, _CH)])


@functools.lru_cache(maxsize=None)
def _sc_kernels():
    mesh = plsc.VectorSubcoreMesh(core_axis_name="c", subcore_axis_name="s")
    segsum_h = pl.kernel(
        _sc_segsum_h_body,
        out_type=jax.ShapeDtypeStruct((_NC * _NPAD, _D), jnp.float32),
        mesh=mesh,
        scratch_types=(
            [pltpu.VMEM((_CH, _D), jnp.float32) for _ in range(_NR)]
            + [pltpu.VMEM((_CH,), jnp.int32) for _ in range(_NI)]
            + [pltpu.VMEM((_CH,), jnp.int32) for _ in range(_NI)]
            + [pltpu.VMEM_SHARED((_NPAD, _D), jnp.float32)]
            + [pltpu.SemaphoreType.DMA for _ in range(2 * _NR + _NI)]
        ),
    )
    segsum_he = pl.kernel(
        _sc_segsum_he_body,
        out_type=jax.ShapeDtypeStruct((_NC * _NPAD, _D), jnp.float32),
        mesh=mesh,
        scratch_types=(
            [pltpu.VMEM((_CH, _D), jnp.float32) for _ in range(_NR)]
            + [pltpu.VMEM((_CH,), jnp.int32) for _ in range(_NI)]
            + [pltpu.VMEM_SHARED((_NPAD, _D), jnp.float32)]
            + [pltpu.SemaphoreType.DMA for _ in range(2 * _NR + _NI)]
        ),
    )
    deg_hist = pl.kernel(
        _sc_deg_body,
        out_type=jax.ShapeDtypeStruct((_NC * _NPAD, _D), jnp.float32),
        mesh=mesh,
        scratch_types=[
            pltpu.VMEM((_NCHUNK, _CH), jnp.int32),
            pltpu.VMEM((_CH, _D), jnp.float32),
            pltpu.VMEM((_CH, _D), jnp.float32),
            pltpu.VMEM_SHARED((_NPAD, _D), jnp.float32),
        ],
    )
    return segsum_h, segsum_he, deg_hist


_BN = 1024


def _bdot(x16, w16):
    return jnp.dot(x16, w16, preferred_element_type=jnp.float32)


def _lodot(x, w16):
    xh = x.astype(jnp.bfloat16)
    xl = (x - xh.astype(jnp.float32)).astype(jnp.bfloat16)
    return _bdot(xh, w16) + _bdot(xl, w16)


def _tc_update_body(h_ref, m0_ref, m1_ref, e0_ref, e1_ref, d0_ref, d1_ref,
                    w1t_ref, w2t_ref, w3t_ref, wiht_ref, whht_ref,
                    bmsg_ref, bih_ref, bhh_ref, out_ref):
    h = h_ref[...]
    h16 = h.astype(jnp.bfloat16)
    m = m0_ref[...] + m1_ref[...]
    hes = e0_ref[...] + e1_ref[...]
    deg = d0_ref[:, 0:1] + d1_ref[:, 0:1]
    w1t = w1t_ref[...].astype(jnp.bfloat16)
    w2t = w2t_ref[...].astype(jnp.bfloat16)
    w3t = w3t_ref[...].astype(jnp.bfloat16)
    wiht = wiht_ref[...].astype(jnp.bfloat16)
    whht = whht_ref[...].astype(jnp.bfloat16)
    a = (deg * (_bdot(h16, w1t) + bmsg_ref[0])
         + _lodot(m, w2t) + _lodot(hes, w3t))
    gi = _bdot(a.astype(jnp.bfloat16), wiht) + bih_ref[0]
    gh = _bdot(h16, whht) + bhh_ref[0]
    r = jax.nn.sigmoid(gi[:, :_D] + gh[:, :_D])
    z = jax.nn.sigmoid(gi[:, _D:2 * _D] + gh[:, _D:2 * _D])
    n = jnp.tanh(gi[:, 2 * _D:] + r * gh[:, 2 * _D:])
    out_ref[...] = (1.0 - z) * n + z * h


def _tc_update(h, m_acc, he_acc, deg_acc, w1t, w2t, w3t, wiht, whht,
               bmsg, bih, bhh):
    blk = lambda i: (i, 0)
    blk_hi = lambda i: (i + _NPAD // _BN, 0)
    full = lambda shape: pl.BlockSpec(shape, lambda i: (0, 0))
    return pl.pallas_call(
        _tc_update_body,
        grid=(_NPAD // _BN,),
        in_specs=[
            pl.BlockSpec((_BN, _D), blk),
            pl.BlockSpec((_BN, _D), blk),
            pl.BlockSpec((_BN, _D), blk_hi),
            pl.BlockSpec((_BN, _D), blk),
            pl.BlockSpec((_BN, _D), blk_hi),
            pl.BlockSpec((_BN, _D), blk),
            pl.BlockSpec((_BN, _D), blk_hi),
            full((_D, 2 * _D)), full((_D, 2 * _D)), full((_D, 2 * _D)),
            full((2 * _D, 3 * _D)), full((_D, 3 * _D)),
            full((1, 2 * _D)), full((1, 3 * _D)), full((1, 3 * _D)),
        ],
        out_specs=pl.BlockSpec((_BN, _D), blk),
        out_shape=jax.ShapeDtypeStruct((_NPAD, _D), jnp.float32),
    )(h, m_acc, m_acc, he_acc, he_acc, deg_acc, deg_acc,
      w1t, w2t, w3t, wiht, whht, bmsg, bih, bhh)


def kernel(hv, edge_index, he, W_msg, b_msg, W_ih, W_hh, b_ih, b_hh):
    src = edge_index[0]
    dst = edge_index[1]
    dst3 = dst.reshape(_NW, _NCHUNK, _CH)
    zrows = jnp.zeros((_CH, _D), jnp.float32)
    ones_v = jnp.ones((_CH, _D), jnp.float32)

    segsum_h, segsum_he, deg_hist = _sc_kernels()
    he_bf = he.astype(jnp.bfloat16).astype(jnp.float32)
    he_acc = segsum_he(he_bf, dst, zrows)
    deg_acc = deg_hist(dst3, zrows, ones_v)

    h = jnp.pad(hv, ((0, _NPAD - _N), (0, 0)))
    for t in range(_NUM_ROUNDS):
        h_bf = h.astype(jnp.bfloat16).astype(jnp.float32)
        m_acc = segsum_h(h_bf, src, dst, zrows)
        h = _tc_update(
            h, m_acc, he_acc, deg_acc,
            W_msg[t, :, :_D].T, W_msg[t, :, _D:2 * _D].T, W_msg[t, :, 2 * _D:].T,
            W_ih[t].T, W_hh[t].T,
            b_msg[t][None], b_ih[t][None], b_hh[t][None])
    return h[:_N]
